# baseline (speedup 1.0000x reference)
import functools

import jax
import jax.numpy as jnp
from jax import lax
from jax.experimental import pallas as pl
from jax.experimental.pallas import tpu as pltpu, tpu_sc as plsc

N = 10000
E = 160000
D_IN = 256
HID = 512
HEADS = 8
D_HEAD = 64
D_EDGE = 16
OUT_DIM = 1

NC, NS, L = 2, 16, 16
NT = NC * NS
EP = 163840
ET = EP // NS
CH = 32
NCH = ET // CH
ACC_W = 144
NPT = N // NS

_mesh = plsc.VectorSubcoreMesh(core_axis_name="c", subcore_axis_name="s")


@functools.partial(
    pl.kernel,
    out_type=jax.ShapeDtypeStruct((4, N, ACC_W), jnp.float32),
    mesh=_mesh,
    compiler_params=pltpu.CompilerParams(use_tc_tiling_on_sc=False),
    scratch_types=[
        pltpu.VMEM((ET,), jnp.int32),
        pltpu.VMEM((ET,), jnp.int32),
        pltpu.VMEM((2, CH), jnp.float32),
        pltpu.VMEM((2, CH), jnp.float32),
        pltpu.VMEM((2, CH), jnp.int32),
        pltpu.VMEM((2, CH), jnp.int32),
        pltpu.VMEM((2, CH), jnp.int32),
        pltpu.VMEM((2, CH, 128), jnp.float32),
        pltpu.VMEM((2, CH, 16), jnp.float32),
        pltpu.VMEM((2, CH, 16), jnp.float32),
        pltpu.VMEM((CH, ACC_W), jnp.float32),
        pltpu.VMEM_SHARED((N, ACC_W), jnp.float32),
        pltpu.SemaphoreType.DMA,
    ],
)
def _edge_kernel(xl_ref, src_ref, dst_ref, logt_ref, ae_ref, parts_ref,
                 srcv, dstv, aeb0, aeb1, idxb, didxb, dstc, gbuf, sbuf,
                 dbuf, scaled, acc, sem):
    c = lax.axis_index("c")
    s = lax.axis_index("s")
    tbase = s * ET
    iota = lax.iota(jnp.int32, L)
    zero = jnp.zeros((L,), jnp.float32)
    den_pat0 = jnp.where(iota == 0, 1.0, 0.0)
    den_pat1 = jnp.where(iota == 1, 1.0, 0.0)

    pltpu.sync_copy(src_ref.at[pl.ds(tbase, ET)], srcv)
    pltpu.sync_copy(dst_ref.at[pl.ds(tbase, ET)], dstv)

    def one_pass(kk, _):
        p = c * 2 + kk
        shift = p * N
        for r in range(CH):
            for f in range(ACC_W // L):
                scaled[r, pl.ds(f * L, L)] = zero
        for q in range(NPT // CH):
            pltpu.sync_copy(
                scaled, acc.at[pl.ds(s * NPT + q * CH, CH)])
        rem = NPT - (NPT // CH) * CH
        if rem:
            pltpu.sync_copy(scaled.at[pl.ds(0, rem)],
                            acc.at[pl.ds(s * NPT + (NPT // CH) * CH, rem)])
        plsc.subcore_barrier()

        def issue(buf, j):
            for l in range(CH // L):
                idxb[buf, pl.ds(l * L, L)] = (
                    srcv[pl.ds(j * CH + l * L, L)] + shift)
                didxb[buf, pl.ds(l * L, L)] = (
                    dstv[pl.ds(j * CH + l * L, L)] + shift)
            pltpu.async_copy(xl_ref.at[idxb.at[buf]], gbuf.at[buf], sem)
            pltpu.async_copy(logt_ref.at[idxb.at[buf]], sbuf.at[buf], sem)
            pltpu.async_copy(logt_ref.at[didxb.at[buf]], dbuf.at[buf], sem)
            off0 = pl.multiple_of(2 * p * EP + tbase + j * CH, 8)
            off1 = pl.multiple_of((2 * p + 1) * EP + tbase + j * CH, 8)
            pltpu.async_copy(ae_ref.at[pl.ds(off0, CH)], aeb0.at[buf], sem)
            pltpu.async_copy(ae_ref.at[pl.ds(off1, CH)], aeb1.at[buf], sem)

        def wait(buf):
            pltpu.make_async_copy(xl_ref.at[idxb.at[buf]], gbuf.at[buf],
                                  sem).wait()
            pltpu.make_async_copy(logt_ref.at[idxb.at[buf]], sbuf.at[buf],
                                  sem).wait()
            pltpu.make_async_copy(logt_ref.at[didxb.at[buf]], dbuf.at[buf],
                                  sem).wait()
            pltpu.make_async_copy(ae_ref.at[pl.ds(0, CH)], aeb0.at[buf],
                                  sem).wait()
            pltpu.make_async_copy(ae_ref.at[pl.ds(0, CH)], aeb1.at[buf],
                                  sem).wait()

        def compute(buf, j):
            for g in range(CH // L):
                a0 = zero
                a1 = zero
                for l in range(L):
                    e = g * L + l
                    srow = sbuf[buf, e, pl.ds(0, L)]
                    drow = dbuf[buf, e, pl.ds(0, L)]
                    lane = (iota == l)
                    a0 = jnp.where(lane, srow[0] + drow[2], a0)
                    a1 = jnp.where(lane, srow[1] + drow[3], a1)
                a0 = a0 + aeb0[buf, pl.ds(g * L, L)]
                a1 = a1 + aeb1[buf, pl.ds(g * L, L)]
                a0 = jnp.where(a0 >= 0.0, a0, 0.2 * a0)
                a1 = jnp.where(a1 >= 0.0, a1, 0.2 * a1)
                ex0 = jnp.exp(a0)
                ex1 = jnp.exp(a1)
                for l in range(L):
                    e = g * L + l
                    w0 = ex0[l]
                    w1 = ex1[l]
                    for f in range(4):
                        scaled[e, pl.ds(f * L, L)] = (
                            gbuf[buf, e, pl.ds(f * L, L)] * w0)
                    for f in range(4, 8):
                        scaled[e, pl.ds(f * L, L)] = (
                            gbuf[buf, e, pl.ds(f * L, L)] * w1)
                    scaled[e, pl.ds(128, L)] = (den_pat0 * w0
                                                + den_pat1 * w1)
            for l in range(CH // L):
                dstc[buf, pl.ds(l * L, L)] = dstv[pl.ds(j * CH + l * L, L)]
            pltpu.sync_copy(scaled, acc.at[dstc.at[buf]], add=True)

        issue(0, 0)
        issue(1, 1)

        def chunk_pair(j2, _):
            ja = 2 * j2
            wait(0)
            compute(0, ja)

            @pl.when(ja + 2 < NCH)
            def _():
                issue(0, ja + 2)

            wait(1)
            compute(1, ja + 1)

            @pl.when(ja + 3 < NCH)
            def _():
                issue(1, ja + 3)

            return 0

        lax.fori_loop(0, NCH // 2, chunk_pair, 0)
        plsc.subcore_barrier()
        pltpu.sync_copy(acc.at[pl.ds(s * NPT, NPT)],
                        parts_ref.at[p, pl.ds(s * NPT, NPT)])
        plsc.subcore_barrier()
        return 0

    lax.fori_loop(0, 2, one_pass, 0)


def _proj_kernel(h_ref, w_ref, b_ref, o_ref):
    hb = h_ref[...].astype(jnp.bfloat16)
    wb = w_ref[...].astype(jnp.bfloat16)
    o_ref[...] = jnp.dot(hb, wb, preferred_element_type=jnp.float32) + b_ref[...]


def _final_proj(h, W_out, b_out):
    blk = 1000
    return pl.pallas_call(
        _proj_kernel,
        grid=(N // blk,),
        in_specs=[
            pl.BlockSpec((blk, HID), lambda i: (i, 0)),
            pl.BlockSpec((HID, OUT_DIM), lambda i: (0, 0)),
            pl.BlockSpec((OUT_DIM,), lambda i: (0,)),
        ],
        out_specs=pl.BlockSpec((blk, OUT_DIM), lambda i: (i, 0)),
        out_shape=jax.ShapeDtypeStruct((N, OUT_DIM), jnp.float32),
    )(h, W_out, b_out)


_BLK = 1000


def _in_proj_kernel(x_ref, w_ref, b_ref, o_ref):
    xb = x_ref[...].astype(jnp.bfloat16)
    wb = w_ref[...].astype(jnp.bfloat16)
    o_ref[...] = jnp.dot(xb, wb, preferred_element_type=jnp.float32) + b_ref[...]


def _in_proj(x, W_in, b_in):
    return pl.pallas_call(
        _in_proj_kernel,
        grid=(N // _BLK,),
        in_specs=[
            pl.BlockSpec((_BLK, D_IN), lambda i: (i, 0)),
            pl.BlockSpec((D_IN, HID), lambda i: (0, 0)),
            pl.BlockSpec((HID,), lambda i: (0,)),
        ],
        out_specs=pl.BlockSpec((_BLK, HID), lambda i: (i, 0)),
        out_shape=jax.ShapeDtypeStruct((N, HID), jnp.float32),
    )(x, W_in, b_in)


def _layer_proj_kernel(h_ref, w_ref, asel_ref, adel_ref,
                       xlf_ref, xln_ref, asrc_ref, adst_ref):
    hb = h_ref[...].astype(jnp.bfloat16)
    wb = w_ref[...].astype(jnp.bfloat16)
    xl = jnp.dot(hb, wb, preferred_element_type=jnp.float32)
    xlf_ref[...] = xl
    xln_ref[...] = xl
    hi = lax.Precision.HIGHEST
    asrc_ref[0] = jnp.dot(xl, asel_ref[0], precision=hi,
                          preferred_element_type=jnp.float32)
    adst_ref[0] = jnp.dot(xl, adel_ref[0], precision=hi,
                          preferred_element_type=jnp.float32)


def _layer_proj(h, W, Asel, Adel):
    nb = N // _BLK
    return pl.pallas_call(
        _layer_proj_kernel,
        grid=(nb, 4),
        in_specs=[
            pl.BlockSpec((_BLK, HID), lambda i, p: (i, 0)),
            pl.BlockSpec((HID, 128), lambda i, p: (0, p)),
            pl.BlockSpec((1, 128, 2), lambda i, p: (p, 0, 0)),
            pl.BlockSpec((1, 128, 2), lambda i, p: (p, 0, 0)),
        ],
        out_specs=[
            pl.BlockSpec((_BLK, 128), lambda i, p: (p * nb + i, 0)),
            pl.BlockSpec((_BLK, 128), lambda i, p: (i, p)),
            pl.BlockSpec((1, _BLK, 2), lambda i, p: (p, i, 0)),
            pl.BlockSpec((1, _BLK, 2), lambda i, p: (p, i, 0)),
        ],
        out_shape=[
            jax.ShapeDtypeStruct((4 * N, 128), jnp.float32),
            jax.ShapeDtypeStruct((N, HID), jnp.float32),
            jax.ShapeDtypeStruct((4, N, 2), jnp.float32),
            jax.ShapeDtypeStruct((4, N, 2), jnp.float32),
        ],
    )(h, W, Asel, Adel)


def _edge_logit_kernel(ea_ref, b_ref, o_ref):
    o_ref[...] = jnp.dot(ea_ref[...], b_ref[...],
                         precision=lax.Precision.HIGHEST,
                         preferred_element_type=jnp.float32)


def _edge_logits(ea_bf, B):
    blk = 2000
    return pl.pallas_call(
        _edge_logit_kernel,
        grid=(E // blk,),
        in_specs=[
            pl.BlockSpec((blk, D_EDGE), lambda i: (i, 0)),
            pl.BlockSpec((D_EDGE, HEADS), lambda i: (0, 0)),
        ],
        out_specs=pl.BlockSpec((blk, HEADS), lambda i: (i, 0)),
        out_shape=jax.ShapeDtypeStruct((E, HEADS), jnp.float32),
    )(ea_bf, B)


def _combine_kernel(num_ref, den_ref, al_ref, aelr_ref, xl_ref, b_ref,
                    o_ref):
    al = al_ref[...] + aelr_ref[...]
    al = jnp.where(al >= 0.0, al, 0.2 * al)
    exl = jnp.exp(al)
    o_ref[...] = jax.nn.relu(
        (num_ref[...] + exl * xl_ref[...])
        / (den_ref[...] + exl) + b_ref[...])


def _combine(num, den_r, al_r, aelr, xl, bias):
    return pl.pallas_call(
        _combine_kernel,
        grid=(N // _BLK,),
        in_specs=[
            pl.BlockSpec((_BLK, HID), lambda i: (i, 0)),
            pl.BlockSpec((_BLK, HID), lambda i: (i, 0)),
            pl.BlockSpec((_BLK, HID), lambda i: (i, 0)),
            pl.BlockSpec((HID,), lambda i: (0,)),
            pl.BlockSpec((_BLK, HID), lambda i: (i, 0)),
            pl.BlockSpec((HID,), lambda i: (0,)),
        ],
        out_specs=pl.BlockSpec((_BLK, HID), lambda i: (i, 0)),
        out_shape=jax.ShapeDtypeStruct((N, HID), jnp.float32),
    )(num, den_r, al_r, aelr, xl, bias)


def _pack_tabs(asrc, adst):
    rows = []
    for p in range(4):
        r = jnp.stack([asrc[:, 2 * p], asrc[:, 2 * p + 1],
                       adst[:, 2 * p], adst[:, 2 * p + 1]], axis=1)
        rows.append(jnp.pad(r, ((0, 0), (0, 12))))
    return jnp.concatenate(rows, axis=0)


def _head_sel(a_vec, heads):
    a = a_vec.reshape(-1)
    sel = jnp.zeros((4, 128, 2), jnp.float32)
    if heads == 1:
        for p in range(4):
            sel = sel.at[p, :64, 0].set(a[p * 128:p * 128 + 64])
            sel = sel.at[p, 64:, 1].set(a[p * 128 + 64:p * 128 + 128])
    else:
        for p in range(4):
            sel = sel.at[p, :64, 0].set(a[2 * p * 64:(2 * p + 1) * 64])
            sel = sel.at[p, 64:, 1].set(a[(2 * p + 1) * 64:(2 * p + 2) * 64])
    return sel


def _gat_layer_sc(h, src_p, dst_p, ae_full, ae_loop, W, a_src, a_dst, bias,
                  heads, d_head):
    Asel = _head_sel(a_src, heads)
    Adel = _head_sel(a_dst, heads)
    xl_flat, xl, asrc_t, adst_t = _layer_proj(h, W, Asel, Adel)
    asrc8 = jnp.transpose(asrc_t, (1, 0, 2)).reshape(N, 8)
    adst8 = jnp.transpose(adst_t, (1, 0, 2)).reshape(N, 8)
    tabs = _pack_tabs(asrc8, adst8)

    if heads == 1:
        asrc1 = asrc8.sum(axis=1, keepdims=True)
        adst1 = adst8.sum(axis=1, keepdims=True)
        asrc8 = jnp.broadcast_to(asrc1, (N, 8))
        adst8 = jnp.broadcast_to(adst1, (N, 8))
        tabs = _pack_tabs(asrc8, adst8)
        ae8 = jnp.broadcast_to(ae_full, (E, 8))
        al8 = asrc8 + adst8 + ae_loop
    else:
        ae8 = ae_full
        al8 = asrc8 + adst8 + ae_loop

    ae_pad = jnp.full((8, EP), -100.0, jnp.float32)
    ae_pad = ae_pad.at[:, :E].set(ae8.T)

    parts = _edge_kernel(xl_flat, src_p, dst_p, tabs, ae_pad.reshape(-1))

    num = jnp.transpose(parts[:, :, :128], (1, 0, 2)).reshape(N, HID)
    den = jnp.transpose(parts[:, :, 128:130], (1, 0, 2)).reshape(N, 8)
    den_r = jnp.repeat(den, 64, axis=1)
    al_r = jnp.repeat(al8, 64, axis=1)
    return _combine(num, den_r, al_r, jnp.zeros((HID,), jnp.float32),
                    xl, bias)


def kernel(x, edge_index, edge_attr, W_in, b_in, W0, We0, as0, ad0, ae0, b0,
           W1, We1, as1, ad1, ae1, b1, W2, We2, as2, ad2, ae2, b2, W_out,
           b_out):
    src, dst = edge_index[0], edge_index[1]
    pad_ids = (jnp.arange(EP - E, dtype=jnp.int32) * 37) % N
    src_p = jnp.concatenate([src, pad_ids])
    dst_p = jnp.concatenate([dst, pad_ids])

    mean_ea = jnp.mean(edge_attr, axis=0)
    ea_bf = edge_attr.astype(jnp.bfloat16).astype(jnp.float32)
    mean_ea_bf = mean_ea.astype(jnp.bfloat16).astype(jnp.float32)

    def edge_B(We, a_edge, heads, d_head):
        Wr = We.astype(jnp.bfloat16).astype(jnp.float32).reshape(
            D_EDGE, heads, d_head)
        return jnp.einsum('khd,hd->kh', Wr, a_edge,
                          precision=lax.Precision.HIGHEST)

    hi = lax.Precision.HIGHEST
    B0 = edge_B(We0, ae0, HEADS, D_HEAD)
    B1 = edge_B(We1, ae1, HEADS, D_HEAD)
    B2 = edge_B(We2, ae2, 1, HID)
    aef0 = _edge_logits(ea_bf, B0)
    aef1 = _edge_logits(ea_bf, B1)
    aef2 = _edge_logits(ea_bf, jnp.broadcast_to(B2, (D_EDGE, HEADS)))[:, :1]
    ael0 = jnp.dot(mean_ea_bf, B0, precision=hi)
    ael1 = jnp.dot(mean_ea_bf, B1, precision=hi)
    ael2 = jnp.dot(mean_ea_bf, B2, precision=hi)

    h = _in_proj(x, W_in, b_in)
    h = _gat_layer_sc(h, src_p, dst_p, aef0, ael0, W0, as0, ad0, b0,
                      HEADS, D_HEAD)
    h = _gat_layer_sc(h, src_p, dst_p, aef1, ael1, W1, as1, ad1, b1,
                      HEADS, D_HEAD)
    h = _gat_layer_sc(h, src_p, dst_p, aef2, ael2, W2, as2, ad2, b2,
                      1, HID)
    return _final_proj(h, W_out, b_out)


# speedup vs baseline: 1.0461x; 1.0461x over previous
import functools

import jax
import jax.numpy as jnp
from jax import lax
from jax.experimental import pallas as pl
from jax.experimental.pallas import tpu as pltpu, tpu_sc as plsc

N = 10000
E = 160000
D_IN = 256
HID = 512
HEADS = 8
D_HEAD = 64
D_EDGE = 16
OUT_DIM = 1

NC, NS, L = 2, 16, 16
NT = NC * NS
EP = 163840
ET = EP // NS
CH = 32
NCH = ET // CH
ACC_W = 144
NPT = N // NS

_mesh = plsc.VectorSubcoreMesh(core_axis_name="c", subcore_axis_name="s")


@functools.partial(
    pl.kernel,
    out_type=jax.ShapeDtypeStruct((4, N, ACC_W), jnp.float32),
    mesh=_mesh,
    compiler_params=pltpu.CompilerParams(use_tc_tiling_on_sc=False),
    scratch_types=[
        pltpu.VMEM((ET,), jnp.int32),
        pltpu.VMEM((ET,), jnp.int32),
        pltpu.VMEM((2, CH), jnp.float32),
        pltpu.VMEM((2, CH), jnp.float32),
        pltpu.VMEM((2, CH), jnp.int32),
        pltpu.VMEM((2, CH), jnp.int32),
        pltpu.VMEM((2, CH), jnp.int32),
        pltpu.VMEM((2, CH, 128), jnp.float32),
        pltpu.VMEM((2, CH, 16), jnp.float32),
        pltpu.VMEM((2, CH, 16), jnp.float32),
        pltpu.VMEM((CH, ACC_W), jnp.float32),
        pltpu.VMEM_SHARED((N, ACC_W), jnp.float32),
        pltpu.SemaphoreType.DMA,
    ],
)
def _edge_kernel(xl_ref, src_ref, dst_ref, logt_ref, ae_ref, parts_ref,
                 srcv, dstv, aeb0, aeb1, idxb, didxb, dstc, gbuf, sbuf,
                 dbuf, scaled, acc, sem):
    c = lax.axis_index("c")
    s = lax.axis_index("s")
    tbase = s * ET
    iota = lax.iota(jnp.int32, L)
    zero = jnp.zeros((L,), jnp.float32)
    den_pat0 = jnp.where(iota == 0, 1.0, 0.0)
    den_pat1 = jnp.where(iota == 1, 1.0, 0.0)

    pltpu.sync_copy(src_ref.at[pl.ds(tbase, ET)], srcv)
    pltpu.sync_copy(dst_ref.at[pl.ds(tbase, ET)], dstv)

    def one_pass(kk, _):
        p = c * 2 + kk
        shift = p * N
        for r in range(CH):
            for f in range(ACC_W // L):
                scaled[r, pl.ds(f * L, L)] = zero
        for q in range(NPT // CH):
            pltpu.sync_copy(
                scaled, acc.at[pl.ds(s * NPT + q * CH, CH)])
        rem = NPT - (NPT // CH) * CH
        if rem:
            pltpu.sync_copy(scaled.at[pl.ds(0, rem)],
                            acc.at[pl.ds(s * NPT + (NPT // CH) * CH, rem)])
        plsc.subcore_barrier()

        def issue(buf, j):
            for l in range(CH // L):
                idxb[buf, pl.ds(l * L, L)] = (
                    srcv[pl.ds(j * CH + l * L, L)] + shift)
                didxb[buf, pl.ds(l * L, L)] = (
                    dstv[pl.ds(j * CH + l * L, L)] + shift)
            pltpu.async_copy(xl_ref.at[idxb.at[buf]], gbuf.at[buf], sem)
            pltpu.async_copy(logt_ref.at[idxb.at[buf]], sbuf.at[buf], sem)
            pltpu.async_copy(logt_ref.at[didxb.at[buf]], dbuf.at[buf], sem)
            off0 = pl.multiple_of(2 * p * EP + tbase + j * CH, 8)
            off1 = pl.multiple_of((2 * p + 1) * EP + tbase + j * CH, 8)
            pltpu.async_copy(ae_ref.at[pl.ds(off0, CH)], aeb0.at[buf], sem)
            pltpu.async_copy(ae_ref.at[pl.ds(off1, CH)], aeb1.at[buf], sem)

        def wait(buf):
            pltpu.make_async_copy(xl_ref.at[idxb.at[buf]], gbuf.at[buf],
                                  sem).wait()
            pltpu.make_async_copy(logt_ref.at[idxb.at[buf]], sbuf.at[buf],
                                  sem).wait()
            pltpu.make_async_copy(logt_ref.at[didxb.at[buf]], dbuf.at[buf],
                                  sem).wait()
            pltpu.make_async_copy(ae_ref.at[pl.ds(0, CH)], aeb0.at[buf],
                                  sem).wait()
            pltpu.make_async_copy(ae_ref.at[pl.ds(0, CH)], aeb1.at[buf],
                                  sem).wait()

        def compute(buf, j):
            for g in range(CH // L):
                a0 = zero
                a1 = zero
                for l in range(L):
                    e = g * L + l
                    srow = sbuf[buf, e, pl.ds(0, L)]
                    drow = dbuf[buf, e, pl.ds(0, L)]
                    lane = (iota == l)
                    a0 = jnp.where(lane, srow[0] + drow[2], a0)
                    a1 = jnp.where(lane, srow[1] + drow[3], a1)
                a0 = a0 + aeb0[buf, pl.ds(g * L, L)]
                a1 = a1 + aeb1[buf, pl.ds(g * L, L)]
                a0 = jnp.where(a0 >= 0.0, a0, 0.2 * a0)
                a1 = jnp.where(a1 >= 0.0, a1, 0.2 * a1)
                ex0 = jnp.exp(a0)
                ex1 = jnp.exp(a1)
                for l in range(L):
                    e = g * L + l
                    w0 = ex0[l]
                    w1 = ex1[l]
                    for f in range(4):
                        scaled[e, pl.ds(f * L, L)] = (
                            gbuf[buf, e, pl.ds(f * L, L)] * w0)
                    for f in range(4, 8):
                        scaled[e, pl.ds(f * L, L)] = (
                            gbuf[buf, e, pl.ds(f * L, L)] * w1)
                    scaled[e, pl.ds(128, L)] = (den_pat0 * w0
                                                + den_pat1 * w1)
            for l in range(CH // L):
                dstc[buf, pl.ds(l * L, L)] = dstv[pl.ds(j * CH + l * L, L)]
            pltpu.sync_copy(scaled, acc.at[dstc.at[buf]], add=True)

        issue(0, 0)
        issue(1, 1)

        def chunk_pair(j2, _):
            ja = 2 * j2
            wait(0)
            compute(0, ja)

            @pl.when(ja + 2 < NCH)
            def _():
                issue(0, ja + 2)

            wait(1)
            compute(1, ja + 1)

            @pl.when(ja + 3 < NCH)
            def _():
                issue(1, ja + 3)

            return 0

        lax.fori_loop(0, NCH // 2, chunk_pair, 0)
        plsc.subcore_barrier()
        pltpu.sync_copy(acc.at[pl.ds(s * NPT, NPT)],
                        parts_ref.at[p, pl.ds(s * NPT, NPT)])
        plsc.subcore_barrier()
        return 0

    lax.fori_loop(0, 2, one_pass, 0)


def _proj_kernel(h_ref, w_ref, b_ref, o_ref):
    hb = h_ref[...].astype(jnp.bfloat16)
    wb = w_ref[...].astype(jnp.bfloat16)
    o_ref[...] = jnp.dot(hb, wb, preferred_element_type=jnp.float32) + b_ref[...]


def _final_proj(h, W_out, b_out):
    blk = 1000
    return pl.pallas_call(
        _proj_kernel,
        grid=(N // blk,),
        in_specs=[
            pl.BlockSpec((blk, HID), lambda i: (i, 0)),
            pl.BlockSpec((HID, OUT_DIM), lambda i: (0, 0)),
            pl.BlockSpec((OUT_DIM,), lambda i: (0,)),
        ],
        out_specs=pl.BlockSpec((blk, OUT_DIM), lambda i: (i, 0)),
        out_shape=jax.ShapeDtypeStruct((N, OUT_DIM), jnp.float32),
    )(h, W_out, b_out)


_BLK = 1000


def _in_proj_kernel(x_ref, w_ref, b_ref, o_ref):
    xb = x_ref[...].astype(jnp.bfloat16)
    wb = w_ref[...].astype(jnp.bfloat16)
    o_ref[...] = jnp.dot(xb, wb, preferred_element_type=jnp.float32) + b_ref[...]


def _in_proj(x, W_in, b_in):
    return pl.pallas_call(
        _in_proj_kernel,
        grid=(N // _BLK,),
        in_specs=[
            pl.BlockSpec((_BLK, D_IN), lambda i: (i, 0)),
            pl.BlockSpec((D_IN, HID), lambda i: (0, 0)),
            pl.BlockSpec((HID,), lambda i: (0,)),
        ],
        out_specs=pl.BlockSpec((_BLK, HID), lambda i: (i, 0)),
        out_shape=jax.ShapeDtypeStruct((N, HID), jnp.float32),
    )(x, W_in, b_in)


def _layer_proj_kernel(h_ref, w_ref, asel_ref, adel_ref,
                       xlf_ref, xln_ref, asrc_ref, adst_ref):
    hb = h_ref[...].astype(jnp.bfloat16)
    wb = w_ref[...].astype(jnp.bfloat16)
    xl = jnp.dot(hb, wb, preferred_element_type=jnp.float32)
    xlf_ref[...] = xl
    xln_ref[...] = xl
    hi = lax.Precision.HIGHEST
    asrc_ref[0] = jnp.dot(xl, asel_ref[0], precision=hi,
                          preferred_element_type=jnp.float32)
    adst_ref[0] = jnp.dot(xl, adel_ref[0], precision=hi,
                          preferred_element_type=jnp.float32)


def _layer_proj(h, W, Asel, Adel):
    nb = N // _BLK
    return pl.pallas_call(
        _layer_proj_kernel,
        grid=(nb, 4),
        in_specs=[
            pl.BlockSpec((_BLK, HID), lambda i, p: (i, 0)),
            pl.BlockSpec((HID, 128), lambda i, p: (0, p)),
            pl.BlockSpec((1, 128, 2), lambda i, p: (p, 0, 0)),
            pl.BlockSpec((1, 128, 2), lambda i, p: (p, 0, 0)),
        ],
        out_specs=[
            pl.BlockSpec((_BLK, 128), lambda i, p: (p * nb + i, 0)),
            pl.BlockSpec((_BLK, 128), lambda i, p: (i, p)),
            pl.BlockSpec((1, _BLK, 2), lambda i, p: (p, i, 0)),
            pl.BlockSpec((1, _BLK, 2), lambda i, p: (p, i, 0)),
        ],
        out_shape=[
            jax.ShapeDtypeStruct((4 * N, 128), jnp.float32),
            jax.ShapeDtypeStruct((N, HID), jnp.float32),
            jax.ShapeDtypeStruct((4, N, 2), jnp.float32),
            jax.ShapeDtypeStruct((4, N, 2), jnp.float32),
        ],
    )(h, W, Asel, Adel)


def _edge_logit_kernel(ea_ref, b_ref, o_ref):
    o_ref[...] = jnp.dot(ea_ref[...], b_ref[...],
                         precision=lax.Precision.HIGHEST,
                         preferred_element_type=jnp.float32)


def _edge_logits(ea_bf, B):
    blk = 2000
    return pl.pallas_call(
        _edge_logit_kernel,
        grid=(E // blk,),
        in_specs=[
            pl.BlockSpec((blk, D_EDGE), lambda i: (i, 0)),
            pl.BlockSpec((D_EDGE, HEADS), lambda i: (0, 0)),
        ],
        out_specs=pl.BlockSpec((blk, HEADS), lambda i: (i, 0)),
        out_shape=jax.ShapeDtypeStruct((E, HEADS), jnp.float32),
    )(ea_bf, B)


def _combine_kernel(p_ref, al_ref, xl_ref, b_ref, o_ref):
    pp = p_ref[0]
    alp = al_ref[0]
    num = pp[:, :128]
    blk = num.shape[0]
    den = jnp.concatenate(
        [jnp.broadcast_to(pp[:, 128:129], (blk, 64)),
         jnp.broadcast_to(pp[:, 129:130], (blk, 64))], axis=1)
    al = jnp.concatenate(
        [jnp.broadcast_to(alp[:, 0:1], (blk, 64)),
         jnp.broadcast_to(alp[:, 1:2], (blk, 64))], axis=1)
    al = jnp.where(al >= 0.0, al, 0.2 * al)
    exl = jnp.exp(al)
    brow = b_ref[pl.ds(pl.program_id(1), 1), :]
    o_ref[...] = jax.nn.relu(
        (num + exl * xl_ref[...]) / (den + exl) + brow)


def _combine(parts, al_t, xl, bias):
    nb = N // _BLK
    return pl.pallas_call(
        _combine_kernel,
        grid=(nb, 4),
        in_specs=[
            pl.BlockSpec((1, _BLK, ACC_W), lambda i, p: (p, i, 0)),
            pl.BlockSpec((1, _BLK, 2), lambda i, p: (p, i, 0)),
            pl.BlockSpec((_BLK, 128), lambda i, p: (i, p)),
            pl.BlockSpec((4, 128), lambda i, p: (0, 0)),
        ],
        out_specs=pl.BlockSpec((_BLK, 128), lambda i, p: (i, p)),
        out_shape=jax.ShapeDtypeStruct((N, HID), jnp.float32),
    )(parts, al_t, xl, bias)


def _pack_tabs(asrc, adst):
    rows = []
    for p in range(4):
        r = jnp.stack([asrc[:, 2 * p], asrc[:, 2 * p + 1],
                       adst[:, 2 * p], adst[:, 2 * p + 1]], axis=1)
        rows.append(jnp.pad(r, ((0, 0), (0, 12))))
    return jnp.concatenate(rows, axis=0)


def _head_sel(a_vec, heads):
    a = a_vec.reshape(-1)
    sel = jnp.zeros((4, 128, 2), jnp.float32)
    if heads == 1:
        for p in range(4):
            sel = sel.at[p, :64, 0].set(a[p * 128:p * 128 + 64])
            sel = sel.at[p, 64:, 1].set(a[p * 128 + 64:p * 128 + 128])
    else:
        for p in range(4):
            sel = sel.at[p, :64, 0].set(a[2 * p * 64:(2 * p + 1) * 64])
            sel = sel.at[p, 64:, 1].set(a[(2 * p + 1) * 64:(2 * p + 2) * 64])
    return sel


def _gat_layer_sc(h, src_p, dst_p, ae_full, ae_loop, W, a_src, a_dst, bias,
                  heads, d_head):
    Asel = _head_sel(a_src, heads)
    Adel = _head_sel(a_dst, heads)
    xl_flat, xl, asrc_t, adst_t = _layer_proj(h, W, Asel, Adel)
    asrc8 = jnp.transpose(asrc_t, (1, 0, 2)).reshape(N, 8)
    adst8 = jnp.transpose(adst_t, (1, 0, 2)).reshape(N, 8)
    tabs = _pack_tabs(asrc8, adst8)

    if heads == 1:
        asrc1 = asrc8.sum(axis=1, keepdims=True)
        adst1 = adst8.sum(axis=1, keepdims=True)
        asrc8 = jnp.broadcast_to(asrc1, (N, 8))
        adst8 = jnp.broadcast_to(adst1, (N, 8))
        tabs = _pack_tabs(asrc8, adst8)
        ae8 = jnp.broadcast_to(ae_full, (E, 8))
        al8 = asrc8 + adst8 + ae_loop
    else:
        ae8 = ae_full
        al8 = asrc8 + adst8 + ae_loop

    ae_pad = jnp.full((8, EP), -100.0, jnp.float32)
    ae_pad = ae_pad.at[:, :E].set(ae8.T)

    parts = _edge_kernel(xl_flat, src_p, dst_p, tabs, ae_pad.reshape(-1))

    al_t = jnp.transpose(al8.reshape(N, 4, 2), (1, 0, 2))
    return _combine(parts, al_t, xl, bias.reshape(4, 128))


def kernel(x, edge_index, edge_attr, W_in, b_in, W0, We0, as0, ad0, ae0, b0,
           W1, We1, as1, ad1, ae1, b1, W2, We2, as2, ad2, ae2, b2, W_out,
           b_out):
    src, dst = edge_index[0], edge_index[1]
    pad_ids = (jnp.arange(EP - E, dtype=jnp.int32) * 37) % N
    src_p = jnp.concatenate([src, pad_ids])
    dst_p = jnp.concatenate([dst, pad_ids])

    mean_ea = jnp.mean(edge_attr, axis=0)
    ea_bf = edge_attr.astype(jnp.bfloat16).astype(jnp.float32)
    mean_ea_bf = mean_ea.astype(jnp.bfloat16).astype(jnp.float32)

    def edge_B(We, a_edge, heads, d_head):
        Wr = We.astype(jnp.bfloat16).astype(jnp.float32).reshape(
            D_EDGE, heads, d_head)
        return jnp.einsum('khd,hd->kh', Wr, a_edge,
                          precision=lax.Precision.HIGHEST)

    hi = lax.Precision.HIGHEST
    B0 = edge_B(We0, ae0, HEADS, D_HEAD)
    B1 = edge_B(We1, ae1, HEADS, D_HEAD)
    B2 = edge_B(We2, ae2, 1, HID)
    aef0 = _edge_logits(ea_bf, B0)
    aef1 = _edge_logits(ea_bf, B1)
    aef2 = _edge_logits(ea_bf, jnp.broadcast_to(B2, (D_EDGE, HEADS)))[:, :1]
    ael0 = jnp.dot(mean_ea_bf, B0, precision=hi)
    ael1 = jnp.dot(mean_ea_bf, B1, precision=hi)
    ael2 = jnp.dot(mean_ea_bf, B2, precision=hi)

    h = _in_proj(x, W_in, b_in)
    h = _gat_layer_sc(h, src_p, dst_p, aef0, ael0, W0, as0, ad0, b0,
                      HEADS, D_HEAD)
    h = _gat_layer_sc(h, src_p, dst_p, aef1, ael1, W1, as1, ad1, b1,
                      HEADS, D_HEAD)
    h = _gat_layer_sc(h, src_p, dst_p, aef2, ael2, W2, as2, ad2, b2,
                      1, HID)
    return _final_proj(h, W_out, b_out)


# speedup vs baseline: 1.0748x; 1.0275x over previous
import functools

import jax
import jax.numpy as jnp
from jax import lax
from jax.experimental import pallas as pl
from jax.experimental.pallas import tpu as pltpu, tpu_sc as plsc

N = 10000
E = 160000
D_IN = 256
HID = 512
HEADS = 8
D_HEAD = 64
D_EDGE = 16
OUT_DIM = 1

NC, NS, L = 2, 16, 16
NT = NC * NS
EP = 163840
ET = EP // NS
CH = 32
NCH = ET // CH
ACC_W = 144
NPT = N // NS

_mesh = plsc.VectorSubcoreMesh(core_axis_name="c", subcore_axis_name="s")


@functools.partial(
    pl.kernel,
    out_type=jax.ShapeDtypeStruct((4, N, ACC_W), jnp.float32),
    mesh=_mesh,
    compiler_params=pltpu.CompilerParams(use_tc_tiling_on_sc=False),
    scratch_types=[
        pltpu.VMEM((ET,), jnp.int32),
        pltpu.VMEM((ET,), jnp.int32),
        pltpu.VMEM((2, CH), jnp.float32),
        pltpu.VMEM((2, CH), jnp.float32),
        pltpu.VMEM((2, CH), jnp.int32),
        pltpu.VMEM((2, CH), jnp.int32),
        pltpu.VMEM((2, CH), jnp.int32),
        pltpu.VMEM((2, CH, 128), jnp.float32),
        pltpu.VMEM((2, CH, 16), jnp.float32),
        pltpu.VMEM((2, CH, 16), jnp.float32),
        pltpu.VMEM((CH, ACC_W), jnp.float32),
        pltpu.VMEM_SHARED((N, ACC_W), jnp.float32),
        pltpu.SemaphoreType.DMA,
    ],
)
def _edge_kernel(xl_ref, src_ref, dst_ref, logt_ref, ae_ref, parts_ref,
                 srcv, dstv, aeb0, aeb1, idxb, didxb, dstc, gbuf, sbuf,
                 dbuf, scaled, acc, sem):
    c = lax.axis_index("c")
    s = lax.axis_index("s")
    tbase = s * ET
    iota = lax.iota(jnp.int32, L)
    zero = jnp.zeros((L,), jnp.float32)
    den_pat0 = jnp.where(iota == 0, 1.0, 0.0)
    den_pat1 = jnp.where(iota == 1, 1.0, 0.0)

    pltpu.sync_copy(src_ref.at[pl.ds(tbase, ET)], srcv)
    pltpu.sync_copy(dst_ref.at[pl.ds(tbase, ET)], dstv)

    def one_pass(kk, _):
        p = c * 2 + kk
        shift = p * N
        for r in range(CH):
            for f in range(ACC_W // L):
                scaled[r, pl.ds(f * L, L)] = zero
        for q in range(NPT // CH):
            pltpu.sync_copy(
                scaled, acc.at[pl.ds(s * NPT + q * CH, CH)])
        rem = NPT - (NPT // CH) * CH
        if rem:
            pltpu.sync_copy(scaled.at[pl.ds(0, rem)],
                            acc.at[pl.ds(s * NPT + (NPT // CH) * CH, rem)])
        plsc.subcore_barrier()

        def issue(buf, j):
            for l in range(CH // L):
                idxb[buf, pl.ds(l * L, L)] = (
                    srcv[pl.ds(j * CH + l * L, L)] + shift)
                didxb[buf, pl.ds(l * L, L)] = (
                    dstv[pl.ds(j * CH + l * L, L)] + shift)
            pltpu.async_copy(xl_ref.at[idxb.at[buf]], gbuf.at[buf], sem)
            pltpu.async_copy(logt_ref.at[idxb.at[buf]], sbuf.at[buf], sem)
            pltpu.async_copy(logt_ref.at[didxb.at[buf]], dbuf.at[buf], sem)
            off0 = pl.multiple_of(2 * p * EP + tbase + j * CH, 8)
            off1 = pl.multiple_of((2 * p + 1) * EP + tbase + j * CH, 8)
            pltpu.async_copy(ae_ref.at[pl.ds(off0, CH)], aeb0.at[buf], sem)
            pltpu.async_copy(ae_ref.at[pl.ds(off1, CH)], aeb1.at[buf], sem)

        def wait(buf):
            pltpu.make_async_copy(xl_ref.at[idxb.at[buf]], gbuf.at[buf],
                                  sem).wait()
            pltpu.make_async_copy(logt_ref.at[idxb.at[buf]], sbuf.at[buf],
                                  sem).wait()
            pltpu.make_async_copy(logt_ref.at[didxb.at[buf]], dbuf.at[buf],
                                  sem).wait()
            pltpu.make_async_copy(ae_ref.at[pl.ds(0, CH)], aeb0.at[buf],
                                  sem).wait()
            pltpu.make_async_copy(ae_ref.at[pl.ds(0, CH)], aeb1.at[buf],
                                  sem).wait()

        def compute(buf, j):
            for g in range(CH // L):
                a0 = zero
                a1 = zero
                for l in range(L):
                    e = g * L + l
                    srow = sbuf[buf, e, pl.ds(0, L)]
                    drow = dbuf[buf, e, pl.ds(0, L)]
                    lane = (iota == l)
                    a0 = jnp.where(lane, srow[0] + drow[2], a0)
                    a1 = jnp.where(lane, srow[1] + drow[3], a1)
                a0 = a0 + aeb0[buf, pl.ds(g * L, L)]
                a1 = a1 + aeb1[buf, pl.ds(g * L, L)]
                a0 = jnp.where(a0 >= 0.0, a0, 0.2 * a0)
                a1 = jnp.where(a1 >= 0.0, a1, 0.2 * a1)
                ex0 = jnp.exp(a0)
                ex1 = jnp.exp(a1)
                for l in range(L):
                    e = g * L + l
                    w0 = ex0[l]
                    w1 = ex1[l]
                    for f in range(4):
                        scaled[e, pl.ds(f * L, L)] = (
                            gbuf[buf, e, pl.ds(f * L, L)] * w0)
                    for f in range(4, 8):
                        scaled[e, pl.ds(f * L, L)] = (
                            gbuf[buf, e, pl.ds(f * L, L)] * w1)
                    scaled[e, pl.ds(128, L)] = (den_pat0 * w0
                                                + den_pat1 * w1)
            for l in range(CH // L):
                dstc[buf, pl.ds(l * L, L)] = dstv[pl.ds(j * CH + l * L, L)]
            pltpu.sync_copy(scaled, acc.at[dstc.at[buf]], add=True)

        issue(0, 0)
        issue(1, 1)

        def chunk_pair(j2, _):
            ja = 2 * j2
            wait(0)
            compute(0, ja)

            @pl.when(ja + 2 < NCH)
            def _():
                issue(0, ja + 2)

            wait(1)
            compute(1, ja + 1)

            @pl.when(ja + 3 < NCH)
            def _():
                issue(1, ja + 3)

            return 0

        lax.fori_loop(0, NCH // 2, chunk_pair, 0)
        plsc.subcore_barrier()
        pltpu.sync_copy(acc.at[pl.ds(s * NPT, NPT)],
                        parts_ref.at[p, pl.ds(s * NPT, NPT)])
        plsc.subcore_barrier()
        return 0

    lax.fori_loop(0, 2, one_pass, 0)


def _proj_kernel(h_ref, w_ref, b_ref, o_ref):
    hb = h_ref[...].astype(jnp.bfloat16)
    wb = w_ref[...].astype(jnp.bfloat16)
    o_ref[...] = jnp.dot(hb, wb, preferred_element_type=jnp.float32) + b_ref[...]


def _final_proj(h, W_out, b_out):
    blk = 1000
    return pl.pallas_call(
        _proj_kernel,
        grid=(N // blk,),
        in_specs=[
            pl.BlockSpec((blk, HID), lambda i: (i, 0)),
            pl.BlockSpec((HID, OUT_DIM), lambda i: (0, 0)),
            pl.BlockSpec((OUT_DIM,), lambda i: (0,)),
        ],
        out_specs=pl.BlockSpec((blk, OUT_DIM), lambda i: (i, 0)),
        out_shape=jax.ShapeDtypeStruct((N, OUT_DIM), jnp.float32),
    )(h, W_out, b_out)


_BLK = 1000


def _in_proj_kernel(x_ref, w_ref, b_ref, o_ref):
    xb = x_ref[...].astype(jnp.bfloat16)
    wb = w_ref[...].astype(jnp.bfloat16)
    o_ref[...] = jnp.dot(xb, wb, preferred_element_type=jnp.float32) + b_ref[...]


def _in_proj(x, W_in, b_in):
    return pl.pallas_call(
        _in_proj_kernel,
        grid=(N // _BLK,),
        in_specs=[
            pl.BlockSpec((_BLK, D_IN), lambda i: (i, 0)),
            pl.BlockSpec((D_IN, HID), lambda i: (0, 0)),
            pl.BlockSpec((HID,), lambda i: (0,)),
        ],
        out_specs=pl.BlockSpec((_BLK, HID), lambda i: (i, 0)),
        out_shape=jax.ShapeDtypeStruct((N, HID), jnp.float32),
    )(x, W_in, b_in)


def _layer_proj_kernel(h_ref, w_ref, asel_ref, adel_ref,
                       xlf_ref, xln_ref, asrc_ref, adst_ref):
    hb = h_ref[...].astype(jnp.bfloat16)
    wb = w_ref[...].astype(jnp.bfloat16)
    xl = jnp.dot(hb, wb, preferred_element_type=jnp.float32)
    xln_ref[...] = xl
    for p in range(4):
        xlf_ref[p] = xl[:, 128 * p:128 * (p + 1)]
    hi = lax.Precision.HIGHEST
    asrc_ref[...] = jnp.dot(xl, asel_ref[...], precision=hi,
                            preferred_element_type=jnp.float32)
    adst_ref[...] = jnp.dot(xl, adel_ref[...], precision=hi,
                            preferred_element_type=jnp.float32)


def _layer_proj(h, W, Asel, Adel):
    nb = N // _BLK
    return pl.pallas_call(
        _layer_proj_kernel,
        grid=(nb,),
        in_specs=[
            pl.BlockSpec((_BLK, HID), lambda i: (i, 0)),
            pl.BlockSpec((HID, HID), lambda i: (0, 0)),
            pl.BlockSpec((HID, 8), lambda i: (0, 0)),
            pl.BlockSpec((HID, 8), lambda i: (0, 0)),
        ],
        out_specs=[
            pl.BlockSpec((4, _BLK, 128), lambda i: (0, i, 0)),
            pl.BlockSpec((_BLK, HID), lambda i: (i, 0)),
            pl.BlockSpec((_BLK, 8), lambda i: (i, 0)),
            pl.BlockSpec((_BLK, 8), lambda i: (i, 0)),
        ],
        out_shape=[
            jax.ShapeDtypeStruct((4, N, 128), jnp.float32),
            jax.ShapeDtypeStruct((N, HID), jnp.float32),
            jax.ShapeDtypeStruct((N, 8), jnp.float32),
            jax.ShapeDtypeStruct((N, 8), jnp.float32),
        ],
    )(h, W, Asel, Adel)


def _edge_logit_kernel(ea_ref, b_ref, o_ref):
    o_ref[...] = jnp.dot(ea_ref[...], b_ref[...],
                         precision=lax.Precision.HIGHEST,
                         preferred_element_type=jnp.float32)


def _edge_logits(ea_bf, B):
    blk = 2000
    return pl.pallas_call(
        _edge_logit_kernel,
        grid=(E // blk,),
        in_specs=[
            pl.BlockSpec((blk, D_EDGE), lambda i: (i, 0)),
            pl.BlockSpec((D_EDGE, HEADS), lambda i: (0, 0)),
        ],
        out_specs=pl.BlockSpec((blk, HEADS), lambda i: (i, 0)),
        out_shape=jax.ShapeDtypeStruct((E, HEADS), jnp.float32),
    )(ea_bf, B)


def _combine_kernel(p_ref, al_ref, xl_ref, b_ref, o_ref):
    pp = p_ref[0]
    alp = al_ref[0]
    num = pp[:, :128]
    blk = num.shape[0]
    den = jnp.concatenate(
        [jnp.broadcast_to(pp[:, 128:129], (blk, 64)),
         jnp.broadcast_to(pp[:, 129:130], (blk, 64))], axis=1)
    al = jnp.concatenate(
        [jnp.broadcast_to(alp[:, 0:1], (blk, 64)),
         jnp.broadcast_to(alp[:, 1:2], (blk, 64))], axis=1)
    al = jnp.where(al >= 0.0, al, 0.2 * al)
    exl = jnp.exp(al)
    brow = b_ref[pl.ds(pl.program_id(1), 1), :]
    o_ref[...] = jax.nn.relu(
        (num + exl * xl_ref[...]) / (den + exl) + brow)


def _combine(parts, al_t, xl, bias):
    nb = N // _BLK
    return pl.pallas_call(
        _combine_kernel,
        grid=(nb, 4),
        in_specs=[
            pl.BlockSpec((1, _BLK, ACC_W), lambda i, p: (p, i, 0)),
            pl.BlockSpec((1, _BLK, 2), lambda i, p: (p, i, 0)),
            pl.BlockSpec((_BLK, 128), lambda i, p: (i, p)),
            pl.BlockSpec((4, 128), lambda i, p: (0, 0)),
        ],
        out_specs=pl.BlockSpec((_BLK, 128), lambda i, p: (i, p)),
        out_shape=jax.ShapeDtypeStruct((N, HID), jnp.float32),
    )(parts, al_t, xl, bias)


def _pack_tabs(asrc, adst):
    rows = []
    for p in range(4):
        r = jnp.stack([asrc[:, 2 * p], asrc[:, 2 * p + 1],
                       adst[:, 2 * p], adst[:, 2 * p + 1]], axis=1)
        rows.append(jnp.pad(r, ((0, 0), (0, 12))))
    return jnp.concatenate(rows, axis=0)


def _head_sel(a_vec):
    a = a_vec.reshape(-1)
    sel = jnp.zeros((HID, 8), jnp.float32)
    for k in range(8):
        sel = sel.at[64 * k:64 * (k + 1), k].set(a[64 * k:64 * (k + 1)])
    return sel


def _gat_layer_sc(h, src_p, dst_p, ae_full, ae_loop, W, a_src, a_dst, bias,
                  heads, d_head):
    Asel = _head_sel(a_src)
    Adel = _head_sel(a_dst)
    xl_flat3, xl, asrc8, adst8 = _layer_proj(h, W, Asel, Adel)
    xl_flat = xl_flat3.reshape(4 * N, 128)

    if heads == 1:
        asrc8 = jnp.broadcast_to(asrc8.sum(axis=1, keepdims=True), (N, 8))
        adst8 = jnp.broadcast_to(adst8.sum(axis=1, keepdims=True), (N, 8))
        ae8 = jnp.broadcast_to(ae_full, (E, 8))
    else:
        ae8 = ae_full
    tabs = _pack_tabs(asrc8, adst8)
    al8 = asrc8 + adst8 + ae_loop

    ae_pad = jnp.full((8, EP), -100.0, jnp.float32)
    ae_pad = ae_pad.at[:, :E].set(ae8.T)

    parts = _edge_kernel(xl_flat, src_p, dst_p, tabs, ae_pad.reshape(-1))

    al_t = jnp.transpose(al8.reshape(N, 4, 2), (1, 0, 2))
    return _combine(parts, al_t, xl, bias.reshape(4, 128))


def kernel(x, edge_index, edge_attr, W_in, b_in, W0, We0, as0, ad0, ae0, b0,
           W1, We1, as1, ad1, ae1, b1, W2, We2, as2, ad2, ae2, b2, W_out,
           b_out):
    src, dst = edge_index[0], edge_index[1]
    pad_ids = (jnp.arange(EP - E, dtype=jnp.int32) * 37) % N
    src_p = jnp.concatenate([src, pad_ids])
    dst_p = jnp.concatenate([dst, pad_ids])

    mean_ea = jnp.mean(edge_attr, axis=0)
    ea_bf = edge_attr.astype(jnp.bfloat16).astype(jnp.float32)
    mean_ea_bf = mean_ea.astype(jnp.bfloat16).astype(jnp.float32)

    def edge_B(We, a_edge, heads, d_head):
        Wr = We.astype(jnp.bfloat16).astype(jnp.float32).reshape(
            D_EDGE, heads, d_head)
        return jnp.einsum('khd,hd->kh', Wr, a_edge,
                          precision=lax.Precision.HIGHEST)

    hi = lax.Precision.HIGHEST
    B0 = edge_B(We0, ae0, HEADS, D_HEAD)
    B1 = edge_B(We1, ae1, HEADS, D_HEAD)
    B2 = edge_B(We2, ae2, 1, HID)
    aef0 = _edge_logits(ea_bf, B0)
    aef1 = _edge_logits(ea_bf, B1)
    aef2 = _edge_logits(ea_bf, jnp.broadcast_to(B2, (D_EDGE, HEADS)))[:, :1]
    ael0 = jnp.dot(mean_ea_bf, B0, precision=hi)
    ael1 = jnp.dot(mean_ea_bf, B1, precision=hi)
    ael2 = jnp.dot(mean_ea_bf, B2, precision=hi)

    h = _in_proj(x, W_in, b_in)
    h = _gat_layer_sc(h, src_p, dst_p, aef0, ael0, W0, as0, ad0, b0,
                      HEADS, D_HEAD)
    h = _gat_layer_sc(h, src_p, dst_p, aef1, ael1, W1, as1, ad1, b1,
                      HEADS, D_HEAD)
    h = _gat_layer_sc(h, src_p, dst_p, aef2, ael2, W2, as2, ad2, b2,
                      1, HID)
    return _final_proj(h, W_out, b_out)


# speedup vs baseline: 1.1008x; 1.0241x over previous
import functools

import jax
import jax.numpy as jnp
from jax import lax
from jax.experimental import pallas as pl
from jax.experimental.pallas import tpu as pltpu, tpu_sc as plsc

N = 10000
E = 160000
D_IN = 256
HID = 512
HEADS = 8
D_HEAD = 64
D_EDGE = 16
OUT_DIM = 1

NC, NS, L = 2, 16, 16
NT = NC * NS
EP = 163840
ET = EP // NS
CH = 32
NCH = ET // CH
ACC_W = 144
NPT = N // NS

_mesh = plsc.VectorSubcoreMesh(core_axis_name="c", subcore_axis_name="s")


@functools.partial(
    pl.kernel,
    out_type=(jax.ShapeDtypeStruct((4, N, 128), jnp.float32),
              jax.ShapeDtypeStruct((4, N, 16), jnp.float32)),
    mesh=_mesh,
    compiler_params=pltpu.CompilerParams(use_tc_tiling_on_sc=False),
    scratch_types=[
        pltpu.VMEM((ET,), jnp.int32),
        pltpu.VMEM((ET,), jnp.int32),
        pltpu.VMEM((2, CH), jnp.float32),
        pltpu.VMEM((2, CH), jnp.float32),
        pltpu.VMEM((2, CH), jnp.int32),
        pltpu.VMEM((2, CH), jnp.int32),
        pltpu.VMEM((2, CH), jnp.int32),
        pltpu.VMEM((2, CH, 128), jnp.float32),
        pltpu.VMEM((2, CH, 16), jnp.float32),
        pltpu.VMEM((2, CH, 16), jnp.float32),
        pltpu.VMEM((CH, ACC_W), jnp.float32),
        pltpu.VMEM_SHARED((N, ACC_W), jnp.float32),
        pltpu.SemaphoreType.DMA,
    ],
)
def _edge_kernel(xl_ref, src_ref, dst_ref, logt_ref, ae_ref, pnum_ref,
                 pden_ref, srcv, dstv, aeb0, aeb1, idxb, didxb, dstc, gbuf,
                 sbuf, dbuf, scaled, acc, sem):
    c = lax.axis_index("c")
    s = lax.axis_index("s")
    tbase = s * ET
    iota = lax.iota(jnp.int32, L)
    zero = jnp.zeros((L,), jnp.float32)
    den_pat0 = jnp.where(iota == 0, 1.0, 0.0)
    den_pat1 = jnp.where(iota == 1, 1.0, 0.0)

    pltpu.sync_copy(src_ref.at[pl.ds(tbase, ET)], srcv)
    pltpu.sync_copy(dst_ref.at[pl.ds(tbase, ET)], dstv)

    def one_pass(kk, _):
        p = c * 2 + kk
        shift = p * N
        for r in range(CH):
            for f in range(ACC_W // L):
                scaled[r, pl.ds(f * L, L)] = zero
        for q in range(NPT // CH):
            pltpu.sync_copy(
                scaled, acc.at[pl.ds(s * NPT + q * CH, CH)])
        rem = NPT - (NPT // CH) * CH
        if rem:
            pltpu.sync_copy(scaled.at[pl.ds(0, rem)],
                            acc.at[pl.ds(s * NPT + (NPT // CH) * CH, rem)])
        plsc.subcore_barrier()

        def issue(buf, j):
            for l in range(CH // L):
                idxb[buf, pl.ds(l * L, L)] = (
                    srcv[pl.ds(j * CH + l * L, L)] + shift)
                didxb[buf, pl.ds(l * L, L)] = (
                    dstv[pl.ds(j * CH + l * L, L)] + shift)
            pltpu.async_copy(xl_ref.at[idxb.at[buf]], gbuf.at[buf], sem)
            pltpu.async_copy(logt_ref.at[idxb.at[buf]], sbuf.at[buf], sem)
            pltpu.async_copy(logt_ref.at[didxb.at[buf]], dbuf.at[buf], sem)
            off0 = pl.multiple_of(2 * p * EP + tbase + j * CH, 8)
            off1 = pl.multiple_of((2 * p + 1) * EP + tbase + j * CH, 8)
            pltpu.async_copy(ae_ref.at[pl.ds(off0, CH)], aeb0.at[buf], sem)
            pltpu.async_copy(ae_ref.at[pl.ds(off1, CH)], aeb1.at[buf], sem)

        def wait(buf):
            pltpu.make_async_copy(xl_ref.at[idxb.at[buf]], gbuf.at[buf],
                                  sem).wait()
            pltpu.make_async_copy(logt_ref.at[idxb.at[buf]], sbuf.at[buf],
                                  sem).wait()
            pltpu.make_async_copy(logt_ref.at[didxb.at[buf]], dbuf.at[buf],
                                  sem).wait()
            pltpu.make_async_copy(ae_ref.at[pl.ds(0, CH)], aeb0.at[buf],
                                  sem).wait()
            pltpu.make_async_copy(ae_ref.at[pl.ds(0, CH)], aeb1.at[buf],
                                  sem).wait()

        def compute(buf, j):
            for g in range(CH // L):
                a0 = zero
                a1 = zero
                for l in range(L):
                    e = g * L + l
                    srow = sbuf[buf, e, pl.ds(0, L)]
                    drow = dbuf[buf, e, pl.ds(0, L)]
                    lane = (iota == l)
                    a0 = jnp.where(lane, srow[0] + drow[2], a0)
                    a1 = jnp.where(lane, srow[1] + drow[3], a1)
                a0 = a0 + aeb0[buf, pl.ds(g * L, L)]
                a1 = a1 + aeb1[buf, pl.ds(g * L, L)]
                a0 = jnp.where(a0 >= 0.0, a0, 0.2 * a0)
                a1 = jnp.where(a1 >= 0.0, a1, 0.2 * a1)
                ex0 = jnp.exp(a0)
                ex1 = jnp.exp(a1)
                for l in range(L):
                    e = g * L + l
                    w0 = ex0[l]
                    w1 = ex1[l]
                    for f in range(4):
                        scaled[e, pl.ds(f * L, L)] = (
                            gbuf[buf, e, pl.ds(f * L, L)] * w0)
                    for f in range(4, 8):
                        scaled[e, pl.ds(f * L, L)] = (
                            gbuf[buf, e, pl.ds(f * L, L)] * w1)
                    scaled[e, pl.ds(128, L)] = (den_pat0 * w0
                                                + den_pat1 * w1)
            for l in range(CH // L):
                dstc[buf, pl.ds(l * L, L)] = dstv[pl.ds(j * CH + l * L, L)]
            pltpu.sync_copy(scaled, acc.at[dstc.at[buf]], add=True)

        issue(0, 0)
        issue(1, 1)

        def chunk_pair(j2, _):
            ja = 2 * j2
            wait(0)
            compute(0, ja)

            @pl.when(ja + 2 < NCH)
            def _():
                issue(0, ja + 2)

            wait(1)
            compute(1, ja + 1)

            @pl.when(ja + 3 < NCH)
            def _():
                issue(1, ja + 3)

            return 0

        lax.fori_loop(0, NCH // 2, chunk_pair, 0)
        plsc.subcore_barrier()
        pltpu.sync_copy(acc.at[pl.ds(s * NPT, NPT), pl.ds(0, 128)],
                        pnum_ref.at[p, pl.ds(s * NPT, NPT)])
        pltpu.sync_copy(acc.at[pl.ds(s * NPT, NPT), pl.ds(128, 16)],
                        pden_ref.at[p, pl.ds(s * NPT, NPT)])
        plsc.subcore_barrier()
        return 0

    lax.fori_loop(0, 2, one_pass, 0)


def _proj_kernel(h_ref, w_ref, b_ref, o_ref):
    hb = h_ref[...].astype(jnp.bfloat16)
    wb = w_ref[...].astype(jnp.bfloat16)
    o_ref[...] = jnp.dot(hb, wb, preferred_element_type=jnp.float32) + b_ref[...]


def _final_proj(h, W_out, b_out):
    blk = 1000
    return pl.pallas_call(
        _proj_kernel,
        grid=(N // blk,),
        in_specs=[
            pl.BlockSpec((blk, HID), lambda i: (i, 0)),
            pl.BlockSpec((HID, OUT_DIM), lambda i: (0, 0)),
            pl.BlockSpec((OUT_DIM,), lambda i: (0,)),
        ],
        out_specs=pl.BlockSpec((blk, OUT_DIM), lambda i: (i, 0)),
        out_shape=jax.ShapeDtypeStruct((N, OUT_DIM), jnp.float32),
    )(h, W_out, b_out)


_BLK = 1000


def _in_proj_kernel(x_ref, w_ref, b_ref, o_ref):
    xb = x_ref[...].astype(jnp.bfloat16)
    wb = w_ref[...].astype(jnp.bfloat16)
    o_ref[...] = jnp.dot(xb, wb, preferred_element_type=jnp.float32) + b_ref[...]


def _in_proj(x, W_in, b_in):
    return pl.pallas_call(
        _in_proj_kernel,
        grid=(N // _BLK,),
        in_specs=[
            pl.BlockSpec((_BLK, D_IN), lambda i: (i, 0)),
            pl.BlockSpec((D_IN, HID), lambda i: (0, 0)),
            pl.BlockSpec((HID,), lambda i: (0,)),
        ],
        out_specs=pl.BlockSpec((_BLK, HID), lambda i: (i, 0)),
        out_shape=jax.ShapeDtypeStruct((N, HID), jnp.float32),
    )(x, W_in, b_in)


def _layer_proj_kernel(h_ref, w_ref, asel_ref, adel_ref,
                       xlf_ref, xln_ref, asrc_ref, adst_ref):
    hb = h_ref[...].astype(jnp.bfloat16)
    wb = w_ref[...].astype(jnp.bfloat16)
    xl = jnp.dot(hb, wb, preferred_element_type=jnp.float32)
    xln_ref[...] = xl
    for p in range(4):
        xlf_ref[p] = xl[:, 128 * p:128 * (p + 1)]
    hi = lax.Precision.HIGHEST
    asrc_ref[...] = jnp.dot(xl, asel_ref[...], precision=hi,
                            preferred_element_type=jnp.float32)
    adst_ref[...] = jnp.dot(xl, adel_ref[...], precision=hi,
                            preferred_element_type=jnp.float32)


def _layer_proj(h, W, Asel, Adel):
    nb = N // _BLK
    return pl.pallas_call(
        _layer_proj_kernel,
        grid=(nb,),
        in_specs=[
            pl.BlockSpec((_BLK, HID), lambda i: (i, 0)),
            pl.BlockSpec((HID, HID), lambda i: (0, 0)),
            pl.BlockSpec((HID, 8), lambda i: (0, 0)),
            pl.BlockSpec((HID, 8), lambda i: (0, 0)),
        ],
        out_specs=[
            pl.BlockSpec((4, _BLK, 128), lambda i: (0, i, 0)),
            pl.BlockSpec((_BLK, HID), lambda i: (i, 0)),
            pl.BlockSpec((_BLK, 8), lambda i: (i, 0)),
            pl.BlockSpec((_BLK, 8), lambda i: (i, 0)),
        ],
        out_shape=[
            jax.ShapeDtypeStruct((4, N, 128), jnp.float32),
            jax.ShapeDtypeStruct((N, HID), jnp.float32),
            jax.ShapeDtypeStruct((N, 8), jnp.float32),
            jax.ShapeDtypeStruct((N, 8), jnp.float32),
        ],
    )(h, W, Asel, Adel)


def _edge_logit_kernel(ea_ref, b_ref, o_ref):
    o_ref[...] = jnp.dot(ea_ref[...], b_ref[...],
                         precision=lax.Precision.HIGHEST,
                         preferred_element_type=jnp.float32)


def _edge_logits(ea_bf, B):
    blk = 2000
    return pl.pallas_call(
        _edge_logit_kernel,
        grid=(E // blk,),
        in_specs=[
            pl.BlockSpec((blk, D_EDGE), lambda i: (i, 0)),
            pl.BlockSpec((D_EDGE, HEADS), lambda i: (0, 0)),
        ],
        out_specs=pl.BlockSpec((blk, HEADS), lambda i: (i, 0)),
        out_shape=jax.ShapeDtypeStruct((E, HEADS), jnp.float32),
    )(ea_bf, B)


def _combine_kernel(pn_ref, pd_ref, al_ref, xl_ref, b_ref, o_ref):
    num = pn_ref[0]
    pd = pd_ref[0]
    alp = al_ref[0]
    blk = num.shape[0]
    den = jnp.concatenate(
        [jnp.broadcast_to(pd[:, 0:1], (blk, 64)),
         jnp.broadcast_to(pd[:, 1:2], (blk, 64))], axis=1)
    al = jnp.concatenate(
        [jnp.broadcast_to(alp[:, 0:1], (blk, 64)),
         jnp.broadcast_to(alp[:, 1:2], (blk, 64))], axis=1)
    al = jnp.where(al >= 0.0, al, 0.2 * al)
    exl = jnp.exp(al)
    brow = b_ref[pl.ds(pl.program_id(1), 1), :]
    o_ref[...] = jax.nn.relu(
        (num + exl * xl_ref[...]) / (den + exl) + brow)


def _combine(pnum, pden, al_t, xl, bias):
    nb = N // _BLK
    return pl.pallas_call(
        _combine_kernel,
        grid=(nb, 4),
        in_specs=[
            pl.BlockSpec((1, _BLK, 128), lambda i, p: (p, i, 0)),
            pl.BlockSpec((1, _BLK, 16), lambda i, p: (p, i, 0)),
            pl.BlockSpec((1, _BLK, 2), lambda i, p: (p, i, 0)),
            pl.BlockSpec((_BLK, 128), lambda i, p: (i, p)),
            pl.BlockSpec((4, 128), lambda i, p: (0, 0)),
        ],
        out_specs=pl.BlockSpec((_BLK, 128), lambda i, p: (i, p)),
        out_shape=jax.ShapeDtypeStruct((N, HID), jnp.float32),
    )(pnum, pden, al_t, xl, bias)


def _pack_tabs(asrc, adst):
    rows = []
    for p in range(4):
        r = jnp.stack([asrc[:, 2 * p], asrc[:, 2 * p + 1],
                       adst[:, 2 * p], adst[:, 2 * p + 1]], axis=1)
        rows.append(jnp.pad(r, ((0, 0), (0, 12))))
    return jnp.concatenate(rows, axis=0)


def _head_sel(a_vec):
    a = a_vec.reshape(-1)
    sel = jnp.zeros((HID, 8), jnp.float32)
    for k in range(8):
        sel = sel.at[64 * k:64 * (k + 1), k].set(a[64 * k:64 * (k + 1)])
    return sel


def _gat_layer_sc(h, src_p, dst_p, ae_full, ae_loop, W, a_src, a_dst, bias,
                  heads, d_head):
    Asel = _head_sel(a_src)
    Adel = _head_sel(a_dst)
    xl_flat3, xl, asrc8, adst8 = _layer_proj(h, W, Asel, Adel)
    xl_flat = xl_flat3.reshape(4 * N, 128)

    if heads == 1:
        asrc8 = jnp.broadcast_to(asrc8.sum(axis=1, keepdims=True), (N, 8))
        adst8 = jnp.broadcast_to(adst8.sum(axis=1, keepdims=True), (N, 8))
        ae8 = jnp.broadcast_to(ae_full, (E, 8))
    else:
        ae8 = ae_full
    tabs = _pack_tabs(asrc8, adst8)
    al8 = asrc8 + adst8 + ae_loop

    ae_pad = jnp.full((8, EP), -100.0, jnp.float32)
    ae_pad = ae_pad.at[:, :E].set(ae8.T)

    pnum, pden = _edge_kernel(xl_flat, src_p, dst_p, tabs,
                              ae_pad.reshape(-1))

    al_t = jnp.transpose(al8.reshape(N, 4, 2), (1, 0, 2))
    return _combine(pnum, pden, al_t, xl, bias.reshape(4, 128))


def kernel(x, edge_index, edge_attr, W_in, b_in, W0, We0, as0, ad0, ae0, b0,
           W1, We1, as1, ad1, ae1, b1, W2, We2, as2, ad2, ae2, b2, W_out,
           b_out):
    src, dst = edge_index[0], edge_index[1]
    pad_ids = (jnp.arange(EP - E, dtype=jnp.int32) * 37) % N
    src_p = jnp.concatenate([src, pad_ids])
    dst_p = jnp.concatenate([dst, pad_ids])

    mean_ea = jnp.mean(edge_attr, axis=0)
    ea_bf = edge_attr.astype(jnp.bfloat16).astype(jnp.float32)
    mean_ea_bf = mean_ea.astype(jnp.bfloat16).astype(jnp.float32)

    def edge_B(We, a_edge, heads, d_head):
        Wr = We.astype(jnp.bfloat16).astype(jnp.float32).reshape(
            D_EDGE, heads, d_head)
        return jnp.einsum('khd,hd->kh', Wr, a_edge,
                          precision=lax.Precision.HIGHEST)

    hi = lax.Precision.HIGHEST
    B0 = edge_B(We0, ae0, HEADS, D_HEAD)
    B1 = edge_B(We1, ae1, HEADS, D_HEAD)
    B2 = edge_B(We2, ae2, 1, HID)
    aef0 = _edge_logits(ea_bf, B0)
    aef1 = _edge_logits(ea_bf, B1)
    aef2 = _edge_logits(ea_bf, jnp.broadcast_to(B2, (D_EDGE, HEADS)))[:, :1]
    ael0 = jnp.dot(mean_ea_bf, B0, precision=hi)
    ael1 = jnp.dot(mean_ea_bf, B1, precision=hi)
    ael2 = jnp.dot(mean_ea_bf, B2, precision=hi)

    h = _in_proj(x, W_in, b_in)
    h = _gat_layer_sc(h, src_p, dst_p, aef0, ael0, W0, as0, ad0, b0,
                      HEADS, D_HEAD)
    h = _gat_layer_sc(h, src_p, dst_p, aef1, ael1, W1, as1, ad1, b1,
                      HEADS, D_HEAD)
    h = _gat_layer_sc(h, src_p, dst_p, aef2, ael2, W2, as2, ad2, b2,
                      1, HID)
    return _final_proj(h, W_out, b_out)
